# baseline (speedup 1.0000x reference)
import functools

import jax
import jax.numpy as jnp
from jax import lax
from jax.experimental import pallas as pl
from jax.experimental.pallas import tpu as pltpu
from jax.experimental.pallas import tpu_sc as plsc

N = 10000
E = 320000
D = 128
NT = 8
NC = 2
NS = 16
NW = NC * NS
CHUNK = 128
NCHUNK = E // CHUNK
ROWS_PER_SUB = N // NS
WCHUNK = 80
PADCHUNK = NW * WCHUNK
NB = 10
BN = N // NB
EB = NCHUNK // NB


def _softplus(x):
    return jnp.maximum(x, 0.0) + jnp.log1p(jnp.exp(-jnp.abs(x)))


def _prologue_body(x_ref, w1_ref, b1_ref, w2_ref, b2_ref, etw_ref,
                   src_ref, et_ref, h_ref, hs_ref, gidx_ref):
    dn = (((1,), (1,)), ((), ()))
    h1 = jax.nn.relu(
        lax.dot_general(x_ref[...], w1_ref[...], dn,
                        preferred_element_type=jnp.float32) + b1_ref[...])
    h = lax.dot_general(h1, w2_ref[...], dn,
                        preferred_element_type=jnp.float32) + b2_ref[...]
    h_ref[...] = h
    w = _softplus(etw_ref[...])
    for t in range(NT):
        hs_ref[t] = w[0, t] * h
    gidx_ref[...] = et_ref[...] * N + src_ref[...]


PEB = 256

_prologue = pl.pallas_call(
    _prologue_body,
    grid=(NB,),
    in_specs=[
        pl.BlockSpec((BN, D), lambda i: (i, 0)),
        pl.BlockSpec((D, D), lambda i: (0, 0)),
        pl.BlockSpec((1, D), lambda i: (0, 0)),
        pl.BlockSpec((D, D), lambda i: (0, 0)),
        pl.BlockSpec((1, D), lambda i: (0, 0)),
        pl.BlockSpec((1, D), lambda i: (0, 0)),
        pl.BlockSpec((PEB, CHUNK), lambda i: (i, 0)),
        pl.BlockSpec((PEB, CHUNK), lambda i: (i, 0)),
    ],
    out_specs=[
        pl.BlockSpec((BN, D), lambda i: (i, 0)),
        pl.BlockSpec((NT, BN, D), lambda i: (0, i, 0)),
        pl.BlockSpec((PEB, CHUNK), lambda i: (i, 0)),
    ],
    out_shape=[
        jax.ShapeDtypeStruct((N, D), jnp.float32),
        jax.ShapeDtypeStruct((NT, N, D), jnp.float32),
        jax.ShapeDtypeStruct((PADCHUNK, CHUNK), jnp.int32),
    ],
    compiler_params=pltpu.CompilerParams(
        dimension_semantics=("parallel",)),
)


_SC_MESH = plsc.VectorSubcoreMesh(core_axis_name="c", subcore_axis_name="s",
                                  num_cores=NC, num_subcores=NS)
WCHUNK = 80
PADCHUNK = NW * WCHUNK
SUBROWS = 640
PIECE = 80


def _worker_ranges():
    c = lax.axis_index("c")
    s = lax.axis_index("s")
    w = c * NS + s
    ntr = jnp.minimum(WCHUNK, jnp.maximum(NCHUNK - WCHUNK * w, 0))
    return c, s, w, ntr


def _acc_pieces(s):
    out = []
    for k in range(SUBROWS // PIECE):
        off = s * SUBROWS + k * PIECE
        out.append((off, off + PIECE <= N))
    return out


def _sc_count_body(dst_hbm, cnt_hbm, dstv, ones_v, zb, acc, ssem):
    c, s, w, ntr = _worker_ranges()

    def init(i, _):
        for cc in range(D // 16):
            zb[i, pl.ds(cc * 16, 16)] = jnp.zeros((16,), jnp.float32)
            ones_v[i, pl.ds(cc * 16, 16)] = jnp.ones((16,), jnp.float32)
        return 0
    lax.fori_loop(0, PIECE, init, 0)

    def init2(i, _):
        for cc in range(D // 16):
            ones_v[PIECE + i, pl.ds(cc * 16, 16)] = jnp.ones((16,),
                                                             jnp.float32)
        return 0
    lax.fori_loop(0, CHUNK - PIECE, init2, 0)
    for off, _g in _acc_pieces(s):
        @pl.when(_g)
        def _(off=off):
            pltpu.sync_copy(zb, acc.at[pl.ds(off, PIECE)])
    plsc.subcore_barrier()

    pltpu.sync_copy(dst_hbm.at[pl.ds(w * WCHUNK, WCHUNK)], dstv)

    DEPTH = 8

    def issue(j):
        pltpu.async_copy(ones_v, acc.at[dstv.at[j]], ssem, add=True)

    def drain_one():
        pltpu.make_async_copy(ones_v, acc.at[dstv.at[0]], ssem).wait()

    def chunk(j, carry):
        drain_one()
        issue(j)
        return carry
    for j in range(DEPTH):
        issue(j)
    lax.fori_loop(DEPTH, ntr, chunk, 0)
    for _ in range(DEPTH):
        drain_one()

    plsc.subcore_barrier()
    for off, _g in _acc_pieces(s):
        @pl.when(_g)
        def _(off=off):
            pltpu.sync_copy(acc.at[pl.ds(off, PIECE)],
                            cnt_hbm.at[pl.ds(c * N + off, PIECE)])


_sc_count = pl.kernel(
    _sc_count_body,
    out_type=jax.ShapeDtypeStruct((NC * N, D), jnp.float32),
    mesh=_SC_MESH,
    scratch_types=[
        pltpu.VMEM((WCHUNK, CHUNK), jnp.int32),
        pltpu.VMEM((CHUNK, D), jnp.float32),
        pltpu.VMEM((PIECE, D), jnp.float32),
        pltpu.VMEM_SHARED((N, D), jnp.float32),
        pltpu.SemaphoreType.DMA,
    ],
)


NBUF = 4
SCH = 64
SNCHUNK = E // SCH
SWCHUNK = 160
SPADCHUNK = NW * SWCHUNK
SHALF = 40
ZROWS = 16


def _sc_msg_body(gidx_hbm, dst_hbm, hs_hbm, m_hbm, idxv, dstv, rowsbuf, zb,
                 acc, *sems):
    c, s, w, _ = _worker_ranges()
    ntr = jnp.minimum(SWCHUNK, jnp.maximum(SNCHUNK - SWCHUNK * w, 0))
    rows = [rowsbuf.at[b] for b in range(NBUF)]
    gsem = list(sems[:NBUF])
    ssem = list(sems[NBUF:])

    def init(i, _):
        for cc in range(D // 16):
            zb[i, pl.ds(cc * 16, 16)] = jnp.zeros((16,), jnp.float32)
        return 0
    lax.fori_loop(0, ZROWS, init, 0)
    for off, _g in _acc_pieces(s):
        @pl.when(_g)
        def _(off=off):
            for p in range(PIECE // ZROWS):
                pltpu.sync_copy(zb, acc.at[pl.ds(off + p * ZROWS, ZROWS)])
    plsc.subcore_barrier()

    def gather(j, b):
        pltpu.async_copy(hs_hbm.at[idxv.at[j]], rows[b], gsem[b])

    def gwait(b):
        pltpu.make_async_copy(hs_hbm.at[idxv.at[0]], rows[b], gsem[b]).wait()

    def scatter(j, b):
        pltpu.async_copy(rows[b], acc.at[dstv.at[j]], ssem[b], add=True)

    def swait(b):
        pltpu.make_async_copy(rows[b], acc.at[dstv.at[0]], ssem[b]).wait()

    for h in range(SWCHUNK // SHALF):
        nh = jnp.clip(ntr - h * SHALF, 0, SHALF)

        @pl.when(nh > 0)
        def _(h=h, nh=nh):
            pltpu.sync_copy(gidx_hbm.at[pl.ds(w * SWCHUNK + h * SHALF,
                                              SHALF)], idxv)
            pltpu.sync_copy(dst_hbm.at[pl.ds(w * SWCHUNK + h * SHALF,
                                             SHALF)], dstv)
            gather(0, 0)
            gather(1, 1)

            def ring(jj, carry):
                j0 = NBUF * jj
                for b in range(NBUF):
                    j = j0 + b
                    gwait(b)
                    scatter(j, b)
                    tgt = (b + 2) % NBUF

                    @pl.when(j + 2 < nh)
                    def _(j=j, tgt=tgt, b=b, jj=jj):
                        if b >= 2:
                            swait(tgt)
                            gather(j + 2, tgt)
                        else:
                            @pl.when(jj > 0)
                            def _():
                                swait(tgt)
                            gather(j + 2, tgt)
                return carry
            lax.fori_loop(0, nh // NBUF, ring, 0)
            for b in range(NBUF):
                swait(b)

    plsc.subcore_barrier()
    for off, _g in _acc_pieces(s):
        @pl.when(_g)
        def _(off=off):
            pltpu.sync_copy(acc.at[pl.ds(off, PIECE)],
                            m_hbm.at[pl.ds(c * N + off, PIECE)])


_sc_msg = pl.kernel(
    _sc_msg_body,
    out_type=jax.ShapeDtypeStruct((NC * N, D), jnp.float32),
    mesh=_SC_MESH,
    scratch_types=(
        [
            pltpu.VMEM((SHALF, SCH), jnp.int32),
            pltpu.VMEM((SHALF, SCH), jnp.int32),
            pltpu.VMEM((NBUF, SCH, D), jnp.float32),
            pltpu.VMEM((ZROWS, D), jnp.float32),
            pltpu.VMEM_SHARED((N, D), jnp.float32),
        ]
        + [pltpu.SemaphoreType.DMA] * (2 * NBUF)
    ),
)


def _gru_common(m0_ref, m1_ref, ic_ref, h_ref, wi_ref, wh_ref,
                bi_ref, bh_ref):
    m = (m0_ref[...] + m1_ref[...]) * ic_ref[...]
    h = h_ref[...]
    dn = (((1,), (1,)), ((), ()))
    gi = lax.dot_general(m, wi_ref[...], dn,
                         preferred_element_type=jnp.float32) + bi_ref[...]
    gh = lax.dot_general(h, wh_ref[...], dn,
                         preferred_element_type=jnp.float32) + bh_ref[...]
    r = jax.nn.sigmoid(gi[:, 0:D] + gh[:, 0:D])
    z = jax.nn.sigmoid(gi[:, D:2 * D] + gh[:, D:2 * D])
    n = jnp.tanh(gi[:, 2 * D:3 * D] + r * gh[:, 2 * D:3 * D])
    return (1.0 - z) * n + z * h


def _gru_hs_body(m0_ref, m1_ref, ic_ref, h_ref, wi_ref, wh_ref,
                 bi_ref, bh_ref, etw_ref, hn_ref, hs_ref):
    hn = _gru_common(m0_ref, m1_ref, ic_ref, h_ref, wi_ref, wh_ref,
                     bi_ref, bh_ref)
    hn_ref[...] = hn
    w = _softplus(etw_ref[...])
    for t in range(NT):
        hs_ref[t] = w[0, t] * hn


def _gru_specs():
    return [
        pl.BlockSpec((BN, D), lambda i: (i, 0)),
        pl.BlockSpec((BN, D), lambda i: (i + NB, 0)),
        pl.BlockSpec((BN, D), lambda i: (i, 0)),
        pl.BlockSpec((BN, D), lambda i: (i, 0)),
        pl.BlockSpec((3 * D, D), lambda i: (0, 0)),
        pl.BlockSpec((3 * D, D), lambda i: (0, 0)),
        pl.BlockSpec((1, 3 * D), lambda i: (0, 0)),
        pl.BlockSpec((1, 3 * D), lambda i: (0, 0)),
        pl.BlockSpec((1, D), lambda i: (0, 0)),
    ]


_gru_hs = pl.pallas_call(
    _gru_hs_body,
    grid=(NB,),
    in_specs=_gru_specs(),
    out_specs=[
        pl.BlockSpec((BN, D), lambda i: (i, 0)),
        pl.BlockSpec((NT, BN, D), lambda i: (0, i, 0)),
    ],
    out_shape=[
        jax.ShapeDtypeStruct((N, D), jnp.float32),
        jax.ShapeDtypeStruct((NT, N, D), jnp.float32),
    ],
    compiler_params=pltpu.CompilerParams(
        dimension_semantics=("parallel",)),
)

def _gru_first_body(m0_ref, m1_ref, c0_ref, c1_ref, h_ref, wi_ref, wh_ref,
                    bi_ref, bh_ref, etw_ref, hn_ref, hs_ref, ic_ref):
    ic_ref[...] = 1.0 / jnp.maximum(c0_ref[...] + c1_ref[...], 1.0)
    hn = _gru_common(m0_ref, m1_ref, ic_ref, h_ref, wi_ref, wh_ref,
                     bi_ref, bh_ref)
    hn_ref[...] = hn
    w = _softplus(etw_ref[...])
    for t in range(NT):
        hs_ref[t] = w[0, t] * hn


_gru_first = pl.pallas_call(
    _gru_first_body,
    grid=(NB,),
    in_specs=[
        pl.BlockSpec((BN, D), lambda i: (i, 0)),
        pl.BlockSpec((BN, D), lambda i: (i + NB, 0)),
        pl.BlockSpec((BN, D), lambda i: (i, 0)),
        pl.BlockSpec((BN, D), lambda i: (i + NB, 0)),
        pl.BlockSpec((BN, D), lambda i: (i, 0)),
        pl.BlockSpec((3 * D, D), lambda i: (0, 0)),
        pl.BlockSpec((3 * D, D), lambda i: (0, 0)),
        pl.BlockSpec((1, 3 * D), lambda i: (0, 0)),
        pl.BlockSpec((1, 3 * D), lambda i: (0, 0)),
        pl.BlockSpec((1, D), lambda i: (0, 0)),
    ],
    out_specs=[
        pl.BlockSpec((BN, D), lambda i: (i, 0)),
        pl.BlockSpec((NT, BN, D), lambda i: (0, i, 0)),
        pl.BlockSpec((BN, D), lambda i: (i, 0)),
    ],
    out_shape=[
        jax.ShapeDtypeStruct((N, D), jnp.float32),
        jax.ShapeDtypeStruct((NT, N, D), jnp.float32),
        jax.ShapeDtypeStruct((N, D), jnp.float32),
    ],
    compiler_params=pltpu.CompilerParams(
        dimension_semantics=("parallel",)),
)


def _final_body(m0_ref, m1_ref, ic_ref, h_ref, wi_ref, wh_ref,
                bi_ref, bh_ref, diff_ref, wf1_ref, bf1_ref, wf2_ref,
                bf2_ref, wimp_ref, out_ref, accv, acca):
    i = pl.program_id(0)

    @pl.when(i == 0)
    def _():
        accv[...] = jnp.zeros_like(accv)
        acca[0, 0] = 0.0

    hn = _gru_common(m0_ref, m1_ref, ic_ref, h_ref, wi_ref, wh_ref,
                     bi_ref, bh_ref)
    cm1 = _softplus(wimp_ref[0, 0])
    ids = BN * i + lax.broadcasted_iota(jnp.int32, (BN, 1), 0)
    hit = jnp.any(ids == diff_ref[...], axis=1, keepdims=True)
    alpha = 1.0 + hit.astype(jnp.float32) * cm1
    accv[...] += jnp.sum(alpha * hn, axis=0, keepdims=True)
    acca[0, 0] += float(BN) + cm1 * jnp.sum(hit.astype(jnp.float32))

    @pl.when(i == NB - 1)
    def _():
        pooled = accv[...] / acca[0, 0]
        dn = (((1,), (1,)), ((), ()))
        t1 = jax.nn.relu(
            lax.dot_general(pooled, wf1_ref[...], dn,
                            preferred_element_type=jnp.float32) + bf1_ref[...])
        o = lax.dot_general(t1, wf2_ref[...], dn,
                            preferred_element_type=jnp.float32) + bf2_ref[...]
        o0 = o[0, 0]
        o1 = o[0, 1]
        mx = jnp.maximum(o0, o1)
        lse = mx + jnp.log(jnp.exp(o0 - mx) + jnp.exp(o1 - mx))
        out_ref[...] = o - lse


_final = pl.pallas_call(
    _final_body,
    grid=(NB,),
    in_specs=[
        pl.BlockSpec((BN, D), lambda i: (i, 0)),
        pl.BlockSpec((BN, D), lambda i: (i + NB, 0)),
        pl.BlockSpec((BN, D), lambda i: (i, 0)),
        pl.BlockSpec((BN, D), lambda i: (i, 0)),
        pl.BlockSpec((3 * D, D), lambda i: (0, 0)),
        pl.BlockSpec((3 * D, D), lambda i: (0, 0)),
        pl.BlockSpec((1, 3 * D), lambda i: (0, 0)),
        pl.BlockSpec((1, 3 * D), lambda i: (0, 0)),
        pl.BlockSpec((1, D), lambda i: (0, 0)),
        pl.BlockSpec((D, D), lambda i: (0, 0)),
        pl.BlockSpec((1, D), lambda i: (0, 0)),
        pl.BlockSpec((D, D), lambda i: (0, 0)),
        pl.BlockSpec((1, D), lambda i: (0, 0)),
        pl.BlockSpec(memory_space=pltpu.SMEM),
    ],
    out_specs=pl.BlockSpec((1, D), lambda i: (0, 0)),
    out_shape=jax.ShapeDtypeStruct((1, D), jnp.float32),
    scratch_shapes=[
        pltpu.VMEM((1, D), jnp.float32),
        pltpu.SMEM((1, 1), jnp.float32),
    ],
    compiler_params=pltpu.CompilerParams(
        dimension_semantics=("arbitrary",)),
)


def kernel(x, edge_index, edge_type, diff_idx, W1, b1, W2, b2, gru_wi, gru_wh,
           gru_bi, gru_bh, etw, Wf1, bf1, Wf2, bf2, w_imp):
    f32 = jnp.float32
    x = x.astype(f32)
    src2 = edge_index[0].astype(jnp.int32).reshape(NCHUNK, CHUNK)
    dst2 = edge_index[1].astype(jnp.int32).reshape(NCHUNK, CHUNK)
    et2 = edge_type.astype(jnp.int32).reshape(NCHUNK, CHUNK)

    b1r = b1.astype(f32).reshape(1, D)
    b2r = b2.astype(f32).reshape(1, D)
    etwp = jnp.pad(etw.astype(f32).reshape(1, NT), ((0, 0), (0, D - NT)))
    bir = gru_bi.astype(f32).reshape(1, 3 * D)
    bhr = gru_bh.astype(f32).reshape(1, 3 * D)
    diffp = jnp.pad(diff_idx.astype(jnp.int32).reshape(1, -1),
                    ((0, 0), (0, D - diff_idx.shape[0])), constant_values=-1)
    wf2p = jnp.pad(Wf2.astype(f32), ((0, D - 2), (0, 0)))
    bf2p = jnp.pad(bf2.astype(f32).reshape(1, 2), ((0, 0), (0, D - 2)))
    bf1r = bf1.astype(f32).reshape(1, D)
    wimp = w_imp.astype(f32).reshape(1, 1)

    pad_rows = PADCHUNK - NCHUNK
    src2p = jnp.pad(src2, ((0, pad_rows), (0, 0)))
    et2p = jnp.pad(et2, ((0, pad_rows), (0, 0)))
    dst_p = jnp.pad(dst2, ((0, pad_rows), (0, 0)))
    h, hs, gidx_p = _prologue(x, W1.astype(f32), b1r, W2.astype(f32), b2r,
                              etwp, src2p, et2p)
    gidx64_p = gidx_p.reshape(SPADCHUNK, SCH)
    dst64_p = dst_p.reshape(SPADCHUNK, SCH)

    cnt = _sc_count(dst_p)

    wi = gru_wi.astype(f32)
    wh = gru_wh.astype(f32)
    mp = _sc_msg(gidx64_p, dst64_p, hs.reshape(NT * N, D))
    h, hs, invc = _gru_first(mp, mp, cnt, cnt, h, wi, wh, bir, bhr, etwp)
    mp = _sc_msg(gidx64_p, dst64_p, hs.reshape(NT * N, D))
    h, hs = _gru_hs(mp, mp, invc, h, wi, wh, bir, bhr, etwp)
    mp = _sc_msg(gidx64_p, dst64_p, hs.reshape(NT * N, D))
    out = _final(mp, mp, invc, h, wi, wh, bir, bhr, diffp,
                 Wf1.astype(f32), bf1r, wf2p, bf2p, wimp)
    return out[:, :2]


# speedup vs baseline: 1.0497x; 1.0497x over previous
import functools

import jax
import jax.numpy as jnp
from jax import lax
from jax.experimental import pallas as pl
from jax.experimental.pallas import tpu as pltpu
from jax.experimental.pallas import tpu_sc as plsc

N = 10000
E = 320000
D = 128
NT = 8
NC = 2
NS = 16
NW = NC * NS
CHUNK = 128
NCHUNK = E // CHUNK
ROWS_PER_SUB = N // NS
NB = 10
BN = N // NB
EB = NCHUNK // NB


def _softplus(x):
    return jnp.maximum(x, 0.0) + jnp.log1p(jnp.exp(-jnp.abs(x)))


def _prologue_body(x_ref, w1_ref, b1_ref, w2_ref, b2_ref, etw_ref,
                   h_ref, hs_ref):
    dn = (((1,), (1,)), ((), ()))
    h1 = jax.nn.relu(
        lax.dot_general(x_ref[...], w1_ref[...], dn,
                        preferred_element_type=jnp.float32) + b1_ref[...])
    h = lax.dot_general(h1, w2_ref[...], dn,
                        preferred_element_type=jnp.float32) + b2_ref[...]
    h_ref[...] = h
    w = _softplus(etw_ref[...])
    for t in range(NT):
        hs_ref[t] = w[0, t] * h


_prologue = pl.pallas_call(
    _prologue_body,
    grid=(NB,),
    in_specs=[
        pl.BlockSpec((BN, D), lambda i: (i, 0)),
        pl.BlockSpec((D, D), lambda i: (0, 0)),
        pl.BlockSpec((1, D), lambda i: (0, 0)),
        pl.BlockSpec((D, D), lambda i: (0, 0)),
        pl.BlockSpec((1, D), lambda i: (0, 0)),
        pl.BlockSpec((1, D), lambda i: (0, 0)),
    ],
    out_specs=[
        pl.BlockSpec((BN, D), lambda i: (i, 0)),
        pl.BlockSpec((NT, BN, D), lambda i: (0, i, 0)),
    ],
    out_shape=[
        jax.ShapeDtypeStruct((N, D), jnp.float32),
        jax.ShapeDtypeStruct((NT, N, D), jnp.float32),
    ],
    compiler_params=pltpu.CompilerParams(
        dimension_semantics=("parallel",)),
)


def _gidx_body(src_ref, et_ref, gidx_ref):
    gidx_ref[...] = et_ref[...] * N + src_ref[...]


_gidx_call = pl.pallas_call(
    _gidx_body,
    out_shape=jax.ShapeDtypeStruct((NCHUNK, CHUNK), jnp.int32),
)


_SC_MESH = plsc.VectorSubcoreMesh(core_axis_name="c", subcore_axis_name="s",
                                  num_cores=NC, num_subcores=NS)
WCHUNK = 80
PADCHUNK = NW * WCHUNK
SUBROWS = 640
PIECE = 80


def _worker_ranges():
    c = lax.axis_index("c")
    s = lax.axis_index("s")
    w = c * NS + s
    ntr = jnp.minimum(WCHUNK, jnp.maximum(NCHUNK - WCHUNK * w, 0))
    return c, s, w, ntr


def _acc_pieces(s):
    out = []
    for k in range(SUBROWS // PIECE):
        off = s * SUBROWS + k * PIECE
        out.append((off, off + PIECE <= N))
    return out


def _sc_count_body(dst_hbm, cnt_hbm, dstv, ones_v, zb, acc, ssem):
    c, s, w, ntr = _worker_ranges()

    def init(i, _):
        for cc in range(D // 16):
            zb[i, pl.ds(cc * 16, 16)] = jnp.zeros((16,), jnp.float32)
            ones_v[i, pl.ds(cc * 16, 16)] = jnp.ones((16,), jnp.float32)
        return 0
    lax.fori_loop(0, PIECE, init, 0)

    def init2(i, _):
        for cc in range(D // 16):
            ones_v[PIECE + i, pl.ds(cc * 16, 16)] = jnp.ones((16,),
                                                             jnp.float32)
        return 0
    lax.fori_loop(0, CHUNK - PIECE, init2, 0)
    for off, _g in _acc_pieces(s):
        @pl.when(_g)
        def _(off=off):
            pltpu.sync_copy(zb, acc.at[pl.ds(off, PIECE)])
    plsc.subcore_barrier()

    pltpu.sync_copy(dst_hbm.at[pl.ds(w * WCHUNK, WCHUNK)], dstv)

    DEPTH = 8

    def issue(j):
        pltpu.async_copy(ones_v, acc.at[dstv.at[j]], ssem, add=True)

    def drain_one():
        pltpu.make_async_copy(ones_v, acc.at[dstv.at[0]], ssem).wait()

    def chunk(j, carry):
        drain_one()
        issue(j)
        return carry
    for j in range(DEPTH):
        issue(j)
    lax.fori_loop(DEPTH, ntr, chunk, 0)
    for _ in range(DEPTH):
        drain_one()

    plsc.subcore_barrier()
    for off, _g in _acc_pieces(s):
        @pl.when(_g)
        def _(off=off):
            pltpu.sync_copy(acc.at[pl.ds(off, PIECE)],
                            cnt_hbm.at[pl.ds(c * N + off, PIECE)])


_sc_count = pl.kernel(
    _sc_count_body,
    out_type=jax.ShapeDtypeStruct((NC * N, D), jnp.float32),
    mesh=_SC_MESH,
    scratch_types=[
        pltpu.VMEM((WCHUNK, CHUNK), jnp.int32),
        pltpu.VMEM((CHUNK, D), jnp.float32),
        pltpu.VMEM((PIECE, D), jnp.float32),
        pltpu.VMEM_SHARED((N, D), jnp.float32),
        pltpu.SemaphoreType.DMA,
    ],
)


NBUF = 4
SCH = 64
SNCHUNK = E // SCH
SWCHUNK = 160
SPADCHUNK = NW * SWCHUNK
SHALF = 40
ZROWS = 16


def _sc_msg_body(gidx_hbm, dst_hbm, hs_hbm, m_hbm, idxv, dstv, rowsbuf, zb,
                 acc, *sems):
    c, s, w, _ = _worker_ranges()
    ntr = jnp.minimum(SWCHUNK, jnp.maximum(SNCHUNK - SWCHUNK * w, 0))
    rows = [rowsbuf.at[b] for b in range(NBUF)]
    gsem = list(sems[:NBUF])
    ssem = list(sems[NBUF:])

    def init(i, _):
        for cc in range(D // 16):
            zb[i, pl.ds(cc * 16, 16)] = jnp.zeros((16,), jnp.float32)
        return 0
    lax.fori_loop(0, ZROWS, init, 0)
    for off, _g in _acc_pieces(s):
        @pl.when(_g)
        def _(off=off):
            for p in range(PIECE // ZROWS):
                pltpu.sync_copy(zb, acc.at[pl.ds(off + p * ZROWS, ZROWS)])
    plsc.subcore_barrier()

    def gather(j, b):
        pltpu.async_copy(hs_hbm.at[idxv.at[j]], rows[b], gsem[b])

    def gwait(b):
        pltpu.make_async_copy(hs_hbm.at[idxv.at[0]], rows[b], gsem[b]).wait()

    def scatter(j, b):
        pltpu.async_copy(rows[b], acc.at[dstv.at[j]], ssem[b], add=True)

    def swait(b):
        pltpu.make_async_copy(rows[b], acc.at[dstv.at[0]], ssem[b]).wait()

    for h in range(SWCHUNK // SHALF):
        nh = jnp.clip(ntr - h * SHALF, 0, SHALF)

        @pl.when(nh > 0)
        def _(h=h, nh=nh):
            pltpu.sync_copy(gidx_hbm.at[pl.ds(w * SWCHUNK + h * SHALF,
                                              SHALF)], idxv)
            pltpu.sync_copy(dst_hbm.at[pl.ds(w * SWCHUNK + h * SHALF,
                                             SHALF)], dstv)
            gather(0, 0)
            gather(1, 1)

            def ring(jj, carry):
                j0 = NBUF * jj
                for b in range(NBUF):
                    j = j0 + b
                    gwait(b)
                    scatter(j, b)
                    tgt = (b + 2) % NBUF

                    @pl.when(j + 2 < nh)
                    def _(j=j, tgt=tgt, b=b, jj=jj):
                        if b >= 2:
                            swait(tgt)
                            gather(j + 2, tgt)
                        else:
                            @pl.when(jj > 0)
                            def _():
                                swait(tgt)
                            gather(j + 2, tgt)
                return carry
            lax.fori_loop(0, nh // NBUF, ring, 0)
            for b in range(NBUF):
                swait(b)

    plsc.subcore_barrier()
    for off, _g in _acc_pieces(s):
        @pl.when(_g)
        def _(off=off):
            pltpu.sync_copy(acc.at[pl.ds(off, PIECE)],
                            m_hbm.at[pl.ds(c * N + off, PIECE)])


_sc_msg = pl.kernel(
    _sc_msg_body,
    out_type=jax.ShapeDtypeStruct((NC * N, D), jnp.float32),
    mesh=_SC_MESH,
    scratch_types=(
        [
            pltpu.VMEM((SHALF, SCH), jnp.int32),
            pltpu.VMEM((SHALF, SCH), jnp.int32),
            pltpu.VMEM((NBUF, SCH, D), jnp.float32),
            pltpu.VMEM((ZROWS, D), jnp.float32),
            pltpu.VMEM_SHARED((N, D), jnp.float32),
        ]
        + [pltpu.SemaphoreType.DMA] * (2 * NBUF)
    ),
)


def _gru_common(m0_ref, m1_ref, ic_ref, h_ref, wi_ref, wh_ref,
                bi_ref, bh_ref):
    m = (m0_ref[...] + m1_ref[...]) * ic_ref[...]
    h = h_ref[...]
    dn = (((1,), (1,)), ((), ()))
    gi = lax.dot_general(m, wi_ref[...], dn,
                         preferred_element_type=jnp.float32) + bi_ref[...]
    gh = lax.dot_general(h, wh_ref[...], dn,
                         preferred_element_type=jnp.float32) + bh_ref[...]
    r = jax.nn.sigmoid(gi[:, 0:D] + gh[:, 0:D])
    z = jax.nn.sigmoid(gi[:, D:2 * D] + gh[:, D:2 * D])
    n = jnp.tanh(gi[:, 2 * D:3 * D] + r * gh[:, 2 * D:3 * D])
    return (1.0 - z) * n + z * h


def _gru_hs_body(m0_ref, m1_ref, ic_ref, h_ref, wi_ref, wh_ref,
                 bi_ref, bh_ref, etw_ref, hn_ref, hs_ref):
    hn = _gru_common(m0_ref, m1_ref, ic_ref, h_ref, wi_ref, wh_ref,
                     bi_ref, bh_ref)
    hn_ref[...] = hn
    w = _softplus(etw_ref[...])
    for t in range(NT):
        hs_ref[t] = w[0, t] * hn


def _gru_specs():
    return [
        pl.BlockSpec((BN, D), lambda i: (i, 0)),
        pl.BlockSpec((BN, D), lambda i: (i + NB, 0)),
        pl.BlockSpec((BN, D), lambda i: (i, 0)),
        pl.BlockSpec((BN, D), lambda i: (i, 0)),
        pl.BlockSpec((3 * D, D), lambda i: (0, 0)),
        pl.BlockSpec((3 * D, D), lambda i: (0, 0)),
        pl.BlockSpec((1, 3 * D), lambda i: (0, 0)),
        pl.BlockSpec((1, 3 * D), lambda i: (0, 0)),
        pl.BlockSpec((1, D), lambda i: (0, 0)),
    ]


_gru_hs = pl.pallas_call(
    _gru_hs_body,
    grid=(NB,),
    in_specs=_gru_specs(),
    out_specs=[
        pl.BlockSpec((BN, D), lambda i: (i, 0)),
        pl.BlockSpec((NT, BN, D), lambda i: (0, i, 0)),
    ],
    out_shape=[
        jax.ShapeDtypeStruct((N, D), jnp.float32),
        jax.ShapeDtypeStruct((NT, N, D), jnp.float32),
    ],
    compiler_params=pltpu.CompilerParams(
        dimension_semantics=("parallel",)),
)

def _invc_body(c0_ref, c1_ref, ic_ref):
    ic_ref[...] = 1.0 / jnp.maximum(c0_ref[...] + c1_ref[...], 1.0)


_invc_call = pl.pallas_call(
    _invc_body,
    grid=(NB,),
    in_specs=[
        pl.BlockSpec((BN, D), lambda i: (i, 0)),
        pl.BlockSpec((BN, D), lambda i: (i + NB, 0)),
    ],
    out_specs=pl.BlockSpec((BN, D), lambda i: (i, 0)),
    out_shape=jax.ShapeDtypeStruct((N, D), jnp.float32),
    compiler_params=pltpu.CompilerParams(
        dimension_semantics=("parallel",)),
)


def _final_body(m0_ref, m1_ref, ic_ref, h_ref, wi_ref, wh_ref,
                bi_ref, bh_ref, diff_ref, wf1_ref, bf1_ref, wf2_ref,
                bf2_ref, wimp_ref, out_ref, accv, acca):
    i = pl.program_id(0)

    @pl.when(i == 0)
    def _():
        accv[...] = jnp.zeros_like(accv)
        acca[0, 0] = 0.0

    hn = _gru_common(m0_ref, m1_ref, ic_ref, h_ref, wi_ref, wh_ref,
                     bi_ref, bh_ref)
    cm1 = _softplus(wimp_ref[0, 0])
    ids = BN * i + lax.broadcasted_iota(jnp.int32, (BN, 1), 0)
    hit = jnp.any(ids == diff_ref[...], axis=1, keepdims=True)
    alpha = 1.0 + hit.astype(jnp.float32) * cm1
    accv[...] += jnp.sum(alpha * hn, axis=0, keepdims=True)
    acca[0, 0] += float(BN) + cm1 * jnp.sum(hit.astype(jnp.float32))

    @pl.when(i == NB - 1)
    def _():
        pooled = accv[...] / acca[0, 0]
        dn = (((1,), (1,)), ((), ()))
        t1 = jax.nn.relu(
            lax.dot_general(pooled, wf1_ref[...], dn,
                            preferred_element_type=jnp.float32) + bf1_ref[...])
        o = lax.dot_general(t1, wf2_ref[...], dn,
                            preferred_element_type=jnp.float32) + bf2_ref[...]
        o0 = o[0, 0]
        o1 = o[0, 1]
        mx = jnp.maximum(o0, o1)
        lse = mx + jnp.log(jnp.exp(o0 - mx) + jnp.exp(o1 - mx))
        out_ref[...] = o - lse


_final = pl.pallas_call(
    _final_body,
    grid=(NB,),
    in_specs=[
        pl.BlockSpec((BN, D), lambda i: (i, 0)),
        pl.BlockSpec((BN, D), lambda i: (i + NB, 0)),
        pl.BlockSpec((BN, D), lambda i: (i, 0)),
        pl.BlockSpec((BN, D), lambda i: (i, 0)),
        pl.BlockSpec((3 * D, D), lambda i: (0, 0)),
        pl.BlockSpec((3 * D, D), lambda i: (0, 0)),
        pl.BlockSpec((1, 3 * D), lambda i: (0, 0)),
        pl.BlockSpec((1, 3 * D), lambda i: (0, 0)),
        pl.BlockSpec((1, D), lambda i: (0, 0)),
        pl.BlockSpec((D, D), lambda i: (0, 0)),
        pl.BlockSpec((1, D), lambda i: (0, 0)),
        pl.BlockSpec((D, D), lambda i: (0, 0)),
        pl.BlockSpec((1, D), lambda i: (0, 0)),
        pl.BlockSpec(memory_space=pltpu.SMEM),
    ],
    out_specs=pl.BlockSpec((1, D), lambda i: (0, 0)),
    out_shape=jax.ShapeDtypeStruct((1, D), jnp.float32),
    scratch_shapes=[
        pltpu.VMEM((1, D), jnp.float32),
        pltpu.SMEM((1, 1), jnp.float32),
    ],
    compiler_params=pltpu.CompilerParams(
        dimension_semantics=("arbitrary",)),
)


def kernel(x, edge_index, edge_type, diff_idx, W1, b1, W2, b2, gru_wi, gru_wh,
           gru_bi, gru_bh, etw, Wf1, bf1, Wf2, bf2, w_imp):
    f32 = jnp.float32
    x = x.astype(f32)
    src2 = edge_index[0].astype(jnp.int32).reshape(NCHUNK, CHUNK)
    dst2 = edge_index[1].astype(jnp.int32).reshape(NCHUNK, CHUNK)
    et2 = edge_type.astype(jnp.int32).reshape(NCHUNK, CHUNK)

    b1r = b1.astype(f32).reshape(1, D)
    b2r = b2.astype(f32).reshape(1, D)
    etwp = jnp.pad(etw.astype(f32).reshape(1, NT), ((0, 0), (0, D - NT)))
    bir = gru_bi.astype(f32).reshape(1, 3 * D)
    bhr = gru_bh.astype(f32).reshape(1, 3 * D)
    diffp = jnp.pad(diff_idx.astype(jnp.int32).reshape(1, -1),
                    ((0, 0), (0, D - diff_idx.shape[0])), constant_values=-1)
    wf2p = jnp.pad(Wf2.astype(f32), ((0, D - 2), (0, 0)))
    bf2p = jnp.pad(bf2.astype(f32).reshape(1, 2), ((0, 0), (0, D - 2)))
    bf1r = bf1.astype(f32).reshape(1, D)
    wimp = w_imp.astype(f32).reshape(1, 1)

    h, hs = _prologue(x, W1.astype(f32), b1r, W2.astype(f32), b2r, etwp)
    gidx2 = _gidx_call(src2, et2)

    pad_rows = PADCHUNK - NCHUNK
    dst_p = jnp.pad(dst2, ((0, pad_rows), (0, 0)))
    spad = SPADCHUNK - SNCHUNK
    gidx64_p = jnp.pad(gidx2.reshape(SNCHUNK, SCH), ((0, spad), (0, 0)))
    dst64_p = jnp.pad(dst2.reshape(SNCHUNK, SCH), ((0, spad), (0, 0)))

    cnt = _sc_count(dst_p)
    invc = _invc_call(cnt, cnt)

    wi = gru_wi.astype(f32)
    wh = gru_wh.astype(f32)
    for it in range(2):
        mp = _sc_msg(gidx64_p, dst64_p, hs.reshape(NT * N, D))
        h, hs = _gru_hs(mp, mp, invc, h, wi, wh, bir, bhr, etwp)
    mp = _sc_msg(gidx64_p, dst64_p, hs.reshape(NT * N, D))
    out = _final(mp, mp, invc, h, wi, wh, bir, bhr, diffp,
                 Wf1.astype(f32), bf1r, wf2p, bf2p, wimp)
    return out[:, :2]
